# baseline (speedup 1.0000x reference)
import functools

import jax
import jax.numpy as jnp
from jax import lax
from jax.experimental import pallas as pl
from jax.experimental.pallas import tpu as pltpu
from jax.experimental.pallas import tpu_sc as plsc

D_IN = 300
D_OUT = 256
TBLK = 2048
CH = 64
NBUF = 4


def _table_body(g_ref, f_ref, wg_ref, wf_ref, b_ref, out_ref):
    dn = (((0,), (0,)), ((), ()))
    acc = lax.dot_general(g_ref[...], wg_ref[...], dn,
                          preferred_element_type=jnp.float32)
    acc = acc + lax.dot_general(f_ref[...], wf_ref[...], dn,
                                preferred_element_type=jnp.float32)
    out_ref[...] = acc + b_ref[...]


def _build_table(glove_t, fast_t, wg_t, wf_t, bias):
    V = glove_t.shape[1]
    grid = pl.cdiv(V, TBLK)
    return pl.pallas_call(
        _table_body,
        grid=(grid,),
        in_specs=[
            pl.BlockSpec((D_IN, TBLK), lambda i: (0, i)),
            pl.BlockSpec((D_IN, TBLK), lambda i: (0, i)),
            pl.BlockSpec((D_IN, D_OUT), lambda i: (0, 0)),
            pl.BlockSpec((D_IN, D_OUT), lambda i: (0, 0)),
            pl.BlockSpec((1, D_OUT), lambda i: (0, 0)),
        ],
        out_specs=pl.BlockSpec((TBLK, D_OUT), lambda i: (i, 0)),
        out_shape=jax.ShapeDtypeStruct((V, D_OUT), jnp.float32),
    )(glove_t, fast_t, wg_t, wf_t, bias)


def _gather(table, idx3):
    info = plsc.get_sparse_core_info()
    NC, NS = info.num_cores, info.num_subcores
    NW = NC * NS
    nw, n_ch, ch = idx3.shape
    assert nw == NW and ch == CH and n_ch % NBUF == 0
    b_per_w = n_ch * CH
    B = NW * b_per_w
    mesh = plsc.VectorSubcoreMesh(core_axis_name="c", subcore_axis_name="s")

    @functools.partial(
        pl.kernel, mesh=mesh,
        out_type=jax.ShapeDtypeStruct((B, D_OUT), jnp.float32),
        scratch_types=[
            pltpu.VMEM((n_ch, CH), jnp.int32),
        ] + [pltpu.VMEM((CH, D_OUT), jnp.float32)] * NBUF
          + [pltpu.SemaphoreType.DMA] * NBUF,
    )
    def k(table_hbm, idx_hbm, out_hbm, idx_v, *rest):
        bufs, sems = rest[:NBUF], rest[NBUF:]
        wid = lax.axis_index("s") * NC + lax.axis_index("c")
        base = pl.multiple_of(wid * b_per_w, CH)
        pltpu.sync_copy(idx_hbm.at[wid], idx_v)
        for p in range(NBUF):
            pltpu.async_copy(table_hbm.at[idx_v.at[p]], bufs[p], sems[p])

        def body(i, carry):
            for p in range(NBUF):
                c = i * NBUF + p
                pltpu.make_async_copy(
                    table_hbm.at[idx_v.at[c]], bufs[p], sems[p]).wait()
                pltpu.sync_copy(
                    bufs[p],
                    out_hbm.at[pl.ds(pl.multiple_of(base + c * CH, CH), CH)])

                @pl.when(c + NBUF < n_ch)
                def _():
                    pltpu.async_copy(
                        table_hbm.at[idx_v.at[c + NBUF]], bufs[p], sems[p])
            return carry

        lax.fori_loop(0, n_ch // NBUF, body, 0)

    return k(table, idx3)


def kernel(ids, glove, fast_text, W_glove, b_glove, W_fast, b_fast):
    B, S = ids.shape
    bias = (b_glove + b_fast).reshape(1, D_OUT)
    table = _build_table(glove.T, fast_text.T, W_glove.T, W_fast.T, bias)
    info = plsc.get_sparse_core_info()
    NW = info.num_cores * info.num_subcores
    tot = B * S
    idx3 = ids.T.reshape(NW, tot // (NW * CH), CH).astype(jnp.int32)
    out = _gather(table, idx3)
    return out.reshape(S, B, D_OUT).transpose(1, 0, 2)


# speedup vs baseline: 1.0057x; 1.0057x over previous
import functools

import jax
import jax.numpy as jnp
from jax import lax
from jax.experimental import pallas as pl
from jax.experimental.pallas import tpu as pltpu
from jax.experimental.pallas import tpu_sc as plsc

D_IN = 300
D_OUT = 256
TBLK = 4096
CH = 64
NBUF = 4


def _table_body(g_ref, f_ref, wg_ref, wf_ref, b_ref, out_ref):
    dn = (((0,), (0,)), ((), ()))
    acc = lax.dot_general(g_ref[...], wg_ref[...], dn,
                          preferred_element_type=jnp.float32)
    acc = acc + lax.dot_general(f_ref[...], wf_ref[...], dn,
                                preferred_element_type=jnp.float32)
    out_ref[...] = acc + b_ref[...]


def _build_table(glove_t, fast_t, wg_t, wf_t, bias):
    V = glove_t.shape[1]
    grid = pl.cdiv(V, TBLK)
    return pl.pallas_call(
        _table_body,
        grid=(grid,),
        in_specs=[
            pl.BlockSpec((D_IN, TBLK), lambda i: (0, i)),
            pl.BlockSpec((D_IN, TBLK), lambda i: (0, i)),
            pl.BlockSpec((D_IN, D_OUT), lambda i: (0, 0)),
            pl.BlockSpec((D_IN, D_OUT), lambda i: (0, 0)),
            pl.BlockSpec((1, D_OUT), lambda i: (0, 0)),
        ],
        out_specs=pl.BlockSpec((TBLK, D_OUT), lambda i: (i, 0)),
        out_shape=jax.ShapeDtypeStruct((V, D_OUT), jnp.float32),
    )(glove_t, fast_t, wg_t, wf_t, bias)


def _gather(table, idx3):
    info = plsc.get_sparse_core_info()
    NC, NS = info.num_cores, info.num_subcores
    NW = NC * NS
    nw, n_ch, ch = idx3.shape
    assert nw == NW and ch == CH and n_ch % NBUF == 0
    b_per_w = n_ch * CH
    B = NW * b_per_w
    mesh = plsc.VectorSubcoreMesh(core_axis_name="c", subcore_axis_name="s")

    @functools.partial(
        pl.kernel, mesh=mesh,
        out_type=jax.ShapeDtypeStruct((B, D_OUT), jnp.float32),
        scratch_types=[
            pltpu.VMEM((n_ch, CH), jnp.int32),
        ] + [pltpu.VMEM((CH, D_OUT), jnp.float32)] * NBUF
          + [pltpu.SemaphoreType.DMA] * NBUF,
    )
    def k(table_hbm, idx_hbm, out_hbm, idx_v, *rest):
        bufs, sems = rest[:NBUF], rest[NBUF:]
        wid = lax.axis_index("s") * NC + lax.axis_index("c")
        base = pl.multiple_of(wid * b_per_w, CH)
        pltpu.sync_copy(idx_hbm.at[wid], idx_v)
        for p in range(NBUF):
            pltpu.async_copy(table_hbm.at[idx_v.at[p]], bufs[p], sems[p])

        def body(i, carry):
            for p in range(NBUF):
                c = i * NBUF + p
                pltpu.make_async_copy(
                    table_hbm.at[idx_v.at[c]], bufs[p], sems[p]).wait()
                pltpu.sync_copy(
                    bufs[p],
                    out_hbm.at[pl.ds(pl.multiple_of(base + c * CH, CH), CH)])

                @pl.when(c + NBUF < n_ch)
                def _():
                    pltpu.async_copy(
                        table_hbm.at[idx_v.at[c + NBUF]], bufs[p], sems[p])
            return carry

        lax.fori_loop(0, n_ch // NBUF, body, 0)

    return k(table, idx3)


def kernel(ids, glove, fast_text, W_glove, b_glove, W_fast, b_fast):
    B, S = ids.shape
    bias = (b_glove + b_fast).reshape(1, D_OUT)
    table = _build_table(glove.T, fast_text.T, W_glove.T, W_fast.T, bias)
    info = plsc.get_sparse_core_info()
    NW = info.num_cores * info.num_subcores
    tot = B * S
    idx3 = ids.T.reshape(NW, tot // (NW * CH), CH).astype(jnp.int32)
    out = _gather(table, idx3)
    return out.reshape(S, B, D_OUT).transpose(1, 0, 2)
